# Initial kernel scaffold; baseline (speedup 1.0000x reference)
import jax
import jax.numpy as jnp
from jax.experimental import pallas as pl


def kernel(x, edge_index, W1, b1, W2, b2):
    raise NotImplementedError("write your pallas kernel here")


# speedup vs baseline: 22.6682x; 22.6682x over previous
import functools

import jax
import jax.numpy as jnp
from jax import lax
from jax.experimental import pallas as pl
from jax.experimental.pallas import tpu as pltpu
from jax.experimental.pallas import tpu_sc as plsc

N_NODES = 10000
NPAD = 10240
N_EDGES = 320000
F = 128
HID = 256
NC = 2
NS = 16
NW = NC * NS
CCH = 125
KSEG = N_EDGES // NW // CCH
KDEG = N_EDGES // NS // CCH
RPT = NPAD // NW
RSUB = 64
RPS = NPAD // NS

_MESH = plsc.VectorSubcoreMesh(core_axis_name="c", subcore_axis_name="s")


def _sc_deg_body(dst_hbm, out_hbm, acc, dstv, onesv, zv):
    c = lax.axis_index("c")
    s = lax.axis_index("s")
    wid = s * NC + c

    def fill(r, _):
        for j in range(8):
            onesv[r, pl.ds(j * 16, 16)] = jnp.ones((16,), jnp.float32)
            zv[r % 64, pl.ds(j * 16, 16)] = jnp.zeros((16,), jnp.float32)
        return 0
    lax.fori_loop(0, CCH, fill, 0)

    def zloop(k, _):
        pltpu.sync_copy(zv, acc.at[pl.ds(s * RPS + k * 64, 64)])
        return 0
    lax.fori_loop(0, RPS // 64, zloop, 0)
    plsc.subcore_barrier()

    pltpu.sync_copy(dst_hbm.at[wid], dstv)

    def hloop(j, _):
        pltpu.sync_copy(onesv, acc.at[dstv.at[j]], add=True)
        return 0
    lax.fori_loop(0, KSEG, hloop, 0)
    plsc.subcore_barrier()

    def outl(k, _):
        rb = s * RPS + k * 64
        pltpu.sync_copy(acc.at[pl.ds(rb, 64)], zv)
        pltpu.sync_copy(zv, out_hbm.at[c, pl.ds(rb, 64)])
        return 0
    lax.fori_loop(0, RPS // 64, outl, 0)


_sc_deg = functools.partial(
    pl.kernel,
    out_type=jax.ShapeDtypeStruct((NC, NPAD, F), jnp.float32),
    mesh=_MESH,
    scratch_types=[
        pltpu.VMEM_SHARED((NPAD, F), jnp.float32),
        pltpu.VMEM((KSEG, CCH), jnp.int32),
        pltpu.VMEM((CCH, F), jnp.float32),
        pltpu.VMEM((64, F), jnp.float32),
    ],
)(_sc_deg_body)


def _sc_segsum_body(src_hbm, dst_hbm, feat_hbm, out_hbm,
                    acc, srcv, dstv, rows, zv, sem):
    c = lax.axis_index("c")
    s = lax.axis_index("s")
    wid = s * NC + c

    def fz(r, _):
        for j in range(8):
            zv[r, pl.ds(j * 16, 16)] = jnp.zeros((16,), jnp.float32)
        return 0
    lax.fori_loop(0, 64, fz, 0)

    def zloop(k, _):
        pltpu.sync_copy(zv, acc.at[pl.ds(s * RPS + k * 64, 64)])
        return 0
    lax.fori_loop(0, RPS // 64, zloop, 0)
    plsc.subcore_barrier()

    pltpu.sync_copy(src_hbm.at[wid], srcv)
    pltpu.sync_copy(dst_hbm.at[wid], dstv)

    def eloop(j, _):
        pltpu.async_copy(feat_hbm.at[srcv.at[j]], rows, sem).wait()
        pltpu.sync_copy(rows, acc.at[dstv.at[j]], add=True)
        return 0
    lax.fori_loop(0, KSEG, eloop, 0)
    plsc.subcore_barrier()

    def outl(k, _):
        rb = s * RPS + k * 64
        pltpu.sync_copy(acc.at[pl.ds(rb, 64)], zv)
        pltpu.sync_copy(zv, out_hbm.at[c, pl.ds(rb, 64)])
        return 0
    lax.fori_loop(0, RPS // 64, outl, 0)


_sc_segsum = functools.partial(
    pl.kernel,
    out_type=jax.ShapeDtypeStruct((NC, NPAD, F), jnp.float32),
    mesh=_MESH,
    scratch_types=[
        pltpu.VMEM_SHARED((NPAD, F), jnp.float32),
        pltpu.VMEM((KSEG, CCH), jnp.int32),
        pltpu.VMEM((KSEG, CCH), jnp.int32),
        pltpu.VMEM((CCH, F), jnp.float32),
        pltpu.VMEM((64, F), jnp.float32),
        pltpu.SemaphoreType.DMA,
    ],
)(_sc_segsum_body)


def _tc_pre_body(degref, xref, xsref, dvref):
    dv = lax.rsqrt(degref[0] + degref[1] + 1.0)
    dvref[...] = dv
    xsref[...] = xref[...] * dv


def _tc_prescale(degp, x_pad):
    BR = 1024
    return pl.pallas_call(
        _tc_pre_body,
        grid=(NPAD // BR,),
        in_specs=[
            pl.BlockSpec((NC, BR, F), lambda i: (0, i, 0)),
            pl.BlockSpec((BR, F), lambda i: (i, 0)),
        ],
        out_specs=[
            pl.BlockSpec((BR, F), lambda i: (i, 0)),
            pl.BlockSpec((BR, F), lambda i: (i, 0)),
        ],
        out_shape=[
            jax.ShapeDtypeStruct((NPAD, F), jnp.float32),
            jax.ShapeDtypeStruct((NPAD, F), jnp.float32),
        ],
    )(degp, x_pad)


def _tc_mm_body(pref, xsref, dvref, w1ref, b1ref, w2ref, href, hwref):
    a = (pref[0] + pref[1] + xsref[...]) * dvref[...]
    h = jnp.dot(a, w1ref[...], preferred_element_type=jnp.float32)
    h = jnp.maximum(h + b1ref[...], 0.0)
    href[...] = h
    hw = jnp.dot(h, w2ref[...], preferred_element_type=jnp.float32)
    hwref[...] = hw * dvref[...]


def _tc_matmul(p, xs, dinv, W1, b1, W2):
    BR = 1024
    return pl.pallas_call(
        _tc_mm_body,
        grid=(NPAD // BR,),
        in_specs=[
            pl.BlockSpec((NC, BR, F), lambda i: (0, i, 0)),
            pl.BlockSpec((BR, F), lambda i: (i, 0)),
            pl.BlockSpec((BR, F), lambda i: (i, 0)),
            pl.BlockSpec((F, HID), lambda i: (0, 0)),
            pl.BlockSpec((1, HID), lambda i: (0, 0)),
            pl.BlockSpec((HID, F), lambda i: (0, 0)),
        ],
        out_specs=[
            pl.BlockSpec((BR, HID), lambda i: (i, 0)),
            pl.BlockSpec((BR, F), lambda i: (i, 0)),
        ],
        out_shape=[
            jax.ShapeDtypeStruct((NPAD, HID), jnp.float32),
            jax.ShapeDtypeStruct((NPAD, F), jnp.float32),
        ],
    )(p, xs, dinv, W1, b1, W2)


def _tc_fin_body(qref, hwref, dvref, b2ref, oref):
    oref[...] = (qref[0] + qref[1] + hwref[...]) * dvref[...] + b2ref[...]


def _tc_final(q, hws, dinv, b2):
    BR = 1024
    return pl.pallas_call(
        _tc_fin_body,
        grid=(NPAD // BR,),
        in_specs=[
            pl.BlockSpec((NC, BR, F), lambda i: (0, i, 0)),
            pl.BlockSpec((BR, F), lambda i: (i, 0)),
            pl.BlockSpec((BR, F), lambda i: (i, 0)),
            pl.BlockSpec((1, F), lambda i: (0, 0)),
        ],
        out_specs=pl.BlockSpec((BR, F), lambda i: (i, 0)),
        out_shape=jax.ShapeDtypeStruct((NPAD, F), jnp.float32),
    )(q, hws, dinv, b2)


def kernel(x, edge_index, W1, b1, W2, b2):
    src = edge_index[0].astype(jnp.int32)
    dst = edge_index[1].astype(jnp.int32)
    x_pad = jnp.pad(x, ((0, NPAD - N_NODES), (0, 0)))
    src_sc = src.reshape(NW, KSEG, CCH)
    dst_sc = dst.reshape(NW, KSEG, CCH)

    degp = _sc_deg(dst_sc)
    xs, dinv = _tc_prescale(degp, x_pad)
    p = _sc_segsum(src_sc, dst_sc, xs)
    h, hws = _tc_matmul(p, xs, dinv, W1, b1.reshape(1, HID), W2)
    q = _sc_segsum(src_sc, dst_sc, hws)
    logits = _tc_final(q, hws, dinv, b2.reshape(1, F))
    return h[:N_NODES], logits[:N_NODES]


# speedup vs baseline: 27.3116x; 1.2048x over previous
import functools

import jax
import jax.numpy as jnp
from jax import lax
from jax.experimental import pallas as pl
from jax.experimental.pallas import tpu as pltpu
from jax.experimental.pallas import tpu_sc as plsc

N_NODES = 10000
NPAD = 10240
N_EDGES = 320000
F = 128
HID = 256
NC = 2
NS = 16
NW = NC * NS
CCH = 125
GIDX = 40
KSEG = N_EDGES // NW // CCH
KDEG = N_EDGES // NS // CCH
RPT = NPAD // NW
RSUB = 64
RPS = NPAD // NS

_MESH = plsc.VectorSubcoreMesh(core_axis_name="c", subcore_axis_name="s")


def _sc_deg_body(dst_hbm, out_hbm, acc, dstv, onesv, zv, sem):
    c = lax.axis_index("c")
    s = lax.axis_index("s")
    wid = s * NC + c

    def fill(r, _):
        for j in range(8):
            onesv[r, pl.ds(j * 16, 16)] = jnp.ones((16,), jnp.float32)
            zv[r % 16, pl.ds(j * 16, 16)] = jnp.zeros((16,), jnp.float32)
        return 0
    lax.fori_loop(0, CCH, fill, 0)

    def zloop(k, _):
        pltpu.sync_copy(zv, acc.at[pl.ds(s * RPS + k * 16, 16)])
        return 0
    lax.fori_loop(0, RPS // 16, zloop, 0)
    plsc.subcore_barrier()

    pltpu.sync_copy(dst_hbm.at[wid], dstv)

    def gloop(grp, _):
        def fire(i, _):
            pltpu.async_copy(onesv, acc.at[dstv.at[grp * 16 + i]], sem,
                             add=True)
            return 0
        lax.fori_loop(0, 16, fire, 0)

        def drain(i, _):
            pltpu.make_async_copy(onesv, acc.at[dstv.at[0]], sem).wait()
            return 0
        lax.fori_loop(0, 16, drain, 0)
        return 0
    lax.fori_loop(0, KSEG // 16, gloop, 0)
    plsc.subcore_barrier()

    def outl(k, _):
        rb = s * RPS + k * 64
        pltpu.sync_copy(acc.at[pl.ds(rb, 64)], out_hbm.at[c, pl.ds(rb, 64)])
        return 0
    lax.fori_loop(0, RPS // 64, outl, 0)


_sc_deg = functools.partial(
    pl.kernel,
    out_type=jax.ShapeDtypeStruct((NC, NPAD, F), jnp.float32),
    mesh=_MESH,
    scratch_types=[
        pltpu.VMEM_SHARED((NPAD, F), jnp.float32),
        pltpu.VMEM((KSEG, CCH), jnp.int32),
        pltpu.VMEM((CCH, F), jnp.float32),
        pltpu.VMEM((16, F), jnp.float32),
        pltpu.SemaphoreType.DMA,
    ],
)(_sc_deg_body)


def _sc_segsum_body(src_hbm, dst_hbm, feat_hbm, out_hbm,
                    acc, srcv, dstv, rows0, rows1, semg0, semg1):
    c = lax.axis_index("c")
    s = lax.axis_index("s")
    wid = s * NC + c

    def fz(r, _):
        for j in range(8):
            rows0[r, pl.ds(j * 16, 16)] = jnp.zeros((16,), jnp.float32)
        return 0
    lax.fori_loop(0, 16, fz, 0)

    def zloop(k, _):
        pltpu.sync_copy(rows0.at[pl.ds(0, 16)],
                        acc.at[pl.ds(s * RPS + k * 16, 16)])
        return 0
    lax.fori_loop(0, RPS // 16, zloop, 0)
    plsc.subcore_barrier()

    for gg in range(KSEG // GIDX):
        pltpu.sync_copy(src_hbm.at[wid, pl.ds(gg * GIDX, GIDX)], srcv)
        pltpu.sync_copy(dst_hbm.at[wid, pl.ds(gg * GIDX, GIDX)], dstv)
        pltpu.async_copy(feat_hbm.at[srcv.at[0]], rows0, semg0)

        def eloop(g, _):
            j0 = 2 * g
            j1 = j0 + 1
            jn = lax.rem(j0 + 2, GIDX)
            pltpu.make_async_copy(feat_hbm.at[srcv.at[j0]], rows0,
                                  semg0).wait()
            pltpu.async_copy(feat_hbm.at[srcv.at[j1]], rows1, semg1)
            pltpu.sync_copy(rows0, acc.at[dstv.at[j0]], add=True)
            pltpu.make_async_copy(feat_hbm.at[srcv.at[j1]], rows1,
                                  semg1).wait()
            pltpu.async_copy(feat_hbm.at[srcv.at[jn]], rows0, semg0)
            pltpu.sync_copy(rows1, acc.at[dstv.at[j1]], add=True)
            return 0
        lax.fori_loop(0, GIDX // 2, eloop, 0)
        pltpu.make_async_copy(feat_hbm.at[srcv.at[0]], rows0, semg0).wait()
    plsc.subcore_barrier()

    def outl(k, _):
        rb = s * RPS + k * 64
        pltpu.sync_copy(acc.at[pl.ds(rb, 64)], out_hbm.at[c, pl.ds(rb, 64)])
        return 0
    lax.fori_loop(0, RPS // 64, outl, 0)


_sc_segsum = functools.partial(
    pl.kernel,
    out_type=jax.ShapeDtypeStruct((NC, NPAD, F), jnp.float32),
    mesh=_MESH,
    scratch_types=[
        pltpu.VMEM_SHARED((NPAD, F), jnp.float32),
        pltpu.VMEM((GIDX, CCH), jnp.int32),
        pltpu.VMEM((GIDX, CCH), jnp.int32),
        pltpu.VMEM((CCH, F), jnp.float32),
        pltpu.VMEM((CCH, F), jnp.float32),
        pltpu.SemaphoreType.DMA,
        pltpu.SemaphoreType.DMA,
    ],
)(_sc_segsum_body)


def _tc_pre_body(degref, xref, xsref, dvref):
    dv = lax.rsqrt(degref[0] + degref[1] + 1.0)
    dvref[...] = dv
    xsref[...] = xref[...] * dv


def _tc_prescale(degp, x_pad):
    BR = 1024
    return pl.pallas_call(
        _tc_pre_body,
        grid=(NPAD // BR,),
        in_specs=[
            pl.BlockSpec((NC, BR, F), lambda i: (0, i, 0)),
            pl.BlockSpec((BR, F), lambda i: (i, 0)),
        ],
        out_specs=[
            pl.BlockSpec((BR, F), lambda i: (i, 0)),
            pl.BlockSpec((BR, F), lambda i: (i, 0)),
        ],
        out_shape=[
            jax.ShapeDtypeStruct((NPAD, F), jnp.float32),
            jax.ShapeDtypeStruct((NPAD, F), jnp.float32),
        ],
    )(degp, x_pad)


def _tc_mm_body(pref, xsref, dvref, w1ref, b1ref, w2ref, href, hwref):
    a = (pref[0] + pref[1] + xsref[...]) * dvref[...]
    h = jnp.dot(a, w1ref[...], preferred_element_type=jnp.float32)
    h = jnp.maximum(h + b1ref[...], 0.0)
    href[...] = h
    hw = jnp.dot(h, w2ref[...], preferred_element_type=jnp.float32)
    hwref[...] = hw * dvref[...]


def _tc_matmul(p, xs, dinv, W1, b1, W2):
    BR = 1024
    return pl.pallas_call(
        _tc_mm_body,
        grid=(NPAD // BR,),
        in_specs=[
            pl.BlockSpec((NC, BR, F), lambda i: (0, i, 0)),
            pl.BlockSpec((BR, F), lambda i: (i, 0)),
            pl.BlockSpec((BR, F), lambda i: (i, 0)),
            pl.BlockSpec((F, HID), lambda i: (0, 0)),
            pl.BlockSpec((1, HID), lambda i: (0, 0)),
            pl.BlockSpec((HID, F), lambda i: (0, 0)),
        ],
        out_specs=[
            pl.BlockSpec((BR, HID), lambda i: (i, 0)),
            pl.BlockSpec((BR, F), lambda i: (i, 0)),
        ],
        out_shape=[
            jax.ShapeDtypeStruct((NPAD, HID), jnp.float32),
            jax.ShapeDtypeStruct((NPAD, F), jnp.float32),
        ],
    )(p, xs, dinv, W1, b1, W2)


def _tc_fin_body(qref, hwref, dvref, b2ref, oref):
    oref[...] = (qref[0] + qref[1] + hwref[...]) * dvref[...] + b2ref[...]


def _tc_final(q, hws, dinv, b2):
    BR = 1024
    return pl.pallas_call(
        _tc_fin_body,
        grid=(NPAD // BR,),
        in_specs=[
            pl.BlockSpec((NC, BR, F), lambda i: (0, i, 0)),
            pl.BlockSpec((BR, F), lambda i: (i, 0)),
            pl.BlockSpec((BR, F), lambda i: (i, 0)),
            pl.BlockSpec((1, F), lambda i: (0, 0)),
        ],
        out_specs=pl.BlockSpec((BR, F), lambda i: (i, 0)),
        out_shape=jax.ShapeDtypeStruct((NPAD, F), jnp.float32),
    )(q, hws, dinv, b2)


def kernel(x, edge_index, W1, b1, W2, b2):
    src = edge_index[0].astype(jnp.int32)
    dst = edge_index[1].astype(jnp.int32)
    x_pad = jnp.pad(x, ((0, NPAD - N_NODES), (0, 0)))
    src_sc = src.reshape(NW, KSEG, CCH)
    dst_sc = dst.reshape(NW, KSEG, CCH)

    degp = _sc_deg(dst_sc)
    xs, dinv = _tc_prescale(degp, x_pad)
    p = _sc_segsum(src_sc, dst_sc, xs)
    h, hws = _tc_matmul(p, xs, dinv, W1, b1.reshape(1, HID), W2)
    q = _sc_segsum(src_sc, dst_sc, hws)
    logits = _tc_final(q, hws, dinv, b2.reshape(1, F))
    return h[:N_NODES], logits[:N_NODES]


# speedup vs baseline: 28.1863x; 1.0320x over previous
import functools

import jax
import jax.numpy as jnp
from jax import lax
from jax.experimental import pallas as pl
from jax.experimental.pallas import tpu as pltpu
from jax.experimental.pallas import tpu_sc as plsc

N_NODES = 10000
NPAD = 10240
N_EDGES = 320000
F = 128
HID = 256
NC = 2
NS = 16
NW = NC * NS
CCH = 125
GIDX = 40
KSEG = N_EDGES // NW // CCH
RPS = NPAD // NS

_MESH = plsc.VectorSubcoreMesh(core_axis_name="c", subcore_axis_name="s")


def _sc_deg_body(dst_hbm, out_hbm, acc, dstv, onesv, zv, sem):
    c = lax.axis_index("c")
    s = lax.axis_index("s")
    wid = s * NC + c

    def fill(r, _):
        for j in range(8):
            onesv[r, pl.ds(j * 16, 16)] = jnp.ones((16,), jnp.float32)
            zv[r % 16, pl.ds(j * 16, 16)] = jnp.zeros((16,), jnp.float32)
        return 0
    lax.fori_loop(0, CCH, fill, 0)

    def zfire(k, _):
        pltpu.async_copy(zv, acc.at[pl.ds(s * RPS + k * 16, 16)], sem)
        return 0
    lax.fori_loop(0, RPS // 16, zfire, 0)

    def zdrain(k, _):
        pltpu.make_async_copy(zv, acc.at[pl.ds(s * RPS, 16)], sem).wait()
        return 0
    lax.fori_loop(0, RPS // 16, zdrain, 0)
    plsc.subcore_barrier()

    pltpu.sync_copy(dst_hbm.at[wid], dstv)

    def gloop(grp, _):
        def fire(i, _):
            pltpu.async_copy(onesv, acc.at[dstv.at[grp * 16 + i]], sem,
                             add=True)
            return 0
        lax.fori_loop(0, 16, fire, 0)

        def drain(i, _):
            pltpu.make_async_copy(onesv, acc.at[dstv.at[0]], sem).wait()
            return 0
        lax.fori_loop(0, 16, drain, 0)
        return 0
    lax.fori_loop(0, KSEG // 16, gloop, 0)
    plsc.subcore_barrier()

    def ofire(k, _):
        rb = s * RPS + k * 64
        pltpu.async_copy(acc.at[pl.ds(rb, 64)], out_hbm.at[c, pl.ds(rb, 64)],
                         sem)
        return 0
    lax.fori_loop(0, RPS // 64, ofire, 0)

    def odrain(k, _):
        rb = s * RPS
        pltpu.make_async_copy(acc.at[pl.ds(rb, 64)],
                              out_hbm.at[c, pl.ds(rb, 64)], sem).wait()
        return 0
    lax.fori_loop(0, RPS // 64, odrain, 0)


_sc_deg = functools.partial(
    pl.kernel,
    out_type=jax.ShapeDtypeStruct((NC, NPAD, F), jnp.float32),
    mesh=_MESH,
    scratch_types=[
        pltpu.VMEM_SHARED((NPAD, F), jnp.float32),
        pltpu.VMEM((KSEG, CCH), jnp.int32),
        pltpu.VMEM((CCH, F), jnp.float32),
        pltpu.VMEM((16, F), jnp.float32),
        pltpu.SemaphoreType.DMA,
    ],
)(_sc_deg_body)


def _sc_segsum_body(src_hbm, dst_hbm, feat_hbm, out_hbm,
                    acc, srcv, dstv, rows0, rows1, sg0, sg1):
    c = lax.axis_index("c")
    s = lax.axis_index("s")
    wid = s * NC + c

    def fz(r, _):
        for j in range(8):
            rows0[r, pl.ds(j * 16, 16)] = jnp.zeros((16,), jnp.float32)
        return 0
    lax.fori_loop(0, 16, fz, 0)
    zsrc = rows0.at[pl.ds(0, 16)]

    def zfire(k, _):
        pltpu.async_copy(zsrc, acc.at[pl.ds(s * RPS + k * 16, 16)], sg0)
        return 0
    lax.fori_loop(0, RPS // 16, zfire, 0)

    def zdrain(k, _):
        pltpu.make_async_copy(zsrc, acc.at[pl.ds(s * RPS, 16)], sg0).wait()
        return 0
    lax.fori_loop(0, RPS // 16, zdrain, 0)
    plsc.subcore_barrier()

    for gg in range(KSEG // GIDX):
        pltpu.sync_copy(src_hbm.at[wid, pl.ds(gg * GIDX, GIDX)], srcv)
        pltpu.sync_copy(dst_hbm.at[wid, pl.ds(gg * GIDX, GIDX)], dstv)
        pltpu.async_copy(feat_hbm.at[srcv.at[0]], rows0, sg0)

        def eloop(g, _):
            j0 = 2 * g
            j1 = j0 + 1
            jn = lax.rem(j0 + 2, GIDX)
            pltpu.make_async_copy(feat_hbm.at[srcv.at[j0]], rows0,
                                  sg0).wait()
            pltpu.async_copy(feat_hbm.at[srcv.at[j1]], rows1, sg1)
            pltpu.sync_copy(rows0, acc.at[dstv.at[j0]], add=True)
            pltpu.make_async_copy(feat_hbm.at[srcv.at[j1]], rows1,
                                  sg1).wait()
            pltpu.async_copy(feat_hbm.at[srcv.at[jn]], rows0, sg0)
            pltpu.sync_copy(rows1, acc.at[dstv.at[j1]], add=True)
            return 0
        lax.fori_loop(0, GIDX // 2, eloop, 0)
        pltpu.make_async_copy(feat_hbm.at[srcv.at[0]], rows0, sg0).wait()
    plsc.subcore_barrier()

    def ofire(k, _):
        rb = s * RPS + k * 64
        pltpu.async_copy(acc.at[pl.ds(rb, 64)], out_hbm.at[c, pl.ds(rb, 64)],
                         sg0)
        return 0
    lax.fori_loop(0, RPS // 64, ofire, 0)

    def odrain(k, _):
        rb = s * RPS
        pltpu.make_async_copy(acc.at[pl.ds(rb, 64)],
                              out_hbm.at[c, pl.ds(rb, 64)], sg0).wait()
        return 0
    lax.fori_loop(0, RPS // 64, odrain, 0)


_sc_segsum = functools.partial(
    pl.kernel,
    out_type=jax.ShapeDtypeStruct((NC, NPAD, F), jnp.float32),
    mesh=_MESH,
    scratch_types=[
        pltpu.VMEM_SHARED((NPAD, F), jnp.float32),
        pltpu.VMEM((GIDX, CCH), jnp.int32),
        pltpu.VMEM((GIDX, CCH), jnp.int32),
        pltpu.VMEM((CCH, F), jnp.float32),
        pltpu.VMEM((CCH, F), jnp.float32),
        pltpu.SemaphoreType.DMA,
        pltpu.SemaphoreType.DMA,
    ],
)(_sc_segsum_body)


def _tc_pre_body(degref, xref, xsref, dvref):
    dv = lax.rsqrt(degref[0] + degref[1] + 1.0)
    dvref[...] = dv
    xsref[...] = xref[...] * dv


def _tc_prescale(degp, x_pad):
    BR = 1024
    return pl.pallas_call(
        _tc_pre_body,
        grid=(NPAD // BR,),
        in_specs=[
            pl.BlockSpec((NC, BR, F), lambda i: (0, i, 0)),
            pl.BlockSpec((BR, F), lambda i: (i, 0)),
        ],
        out_specs=[
            pl.BlockSpec((BR, F), lambda i: (i, 0)),
            pl.BlockSpec((BR, F), lambda i: (i, 0)),
        ],
        out_shape=[
            jax.ShapeDtypeStruct((NPAD, F), jnp.float32),
            jax.ShapeDtypeStruct((NPAD, F), jnp.float32),
        ],
    )(degp, x_pad)


def _tc_mm_body(pref, xsref, dvref, w1ref, b1ref, w2ref, href, hwref):
    a = (pref[0] + pref[1] + xsref[...]) * dvref[...]
    h = jnp.dot(a, w1ref[...], preferred_element_type=jnp.float32)
    h = jnp.maximum(h + b1ref[...], 0.0)
    href[...] = h
    hw = jnp.dot(h, w2ref[...], preferred_element_type=jnp.float32)
    hwref[...] = hw * dvref[...]


def _tc_matmul(p, xs, dinv, W1, b1, W2):
    BR = 1000
    return pl.pallas_call(
        _tc_mm_body,
        grid=(NPAD // BR,),
        in_specs=[
            pl.BlockSpec((NC, BR, F), lambda i: (0, i, 0)),
            pl.BlockSpec((BR, F), lambda i: (i, 0)),
            pl.BlockSpec((BR, F), lambda i: (i, 0)),
            pl.BlockSpec((F, HID), lambda i: (0, 0)),
            pl.BlockSpec((1, HID), lambda i: (0, 0)),
            pl.BlockSpec((HID, F), lambda i: (0, 0)),
        ],
        out_specs=[
            pl.BlockSpec((BR, HID), lambda i: (i, 0)),
            pl.BlockSpec((BR, F), lambda i: (i, 0)),
        ],
        out_shape=[
            jax.ShapeDtypeStruct((N_NODES, HID), jnp.float32),
            jax.ShapeDtypeStruct((NPAD, F), jnp.float32),
        ],
    )(p, xs, dinv, W1, b1, W2)


def _tc_fin_body(qref, hwref, dvref, b2ref, oref):
    oref[...] = (qref[0] + qref[1] + hwref[...]) * dvref[...] + b2ref[...]


def _tc_final(q, hws, dinv, b2):
    BR = 1000
    return pl.pallas_call(
        _tc_fin_body,
        grid=(NPAD // BR,),
        in_specs=[
            pl.BlockSpec((NC, BR, F), lambda i: (0, i, 0)),
            pl.BlockSpec((BR, F), lambda i: (i, 0)),
            pl.BlockSpec((BR, F), lambda i: (i, 0)),
            pl.BlockSpec((1, F), lambda i: (0, 0)),
        ],
        out_specs=pl.BlockSpec((BR, F), lambda i: (i, 0)),
        out_shape=jax.ShapeDtypeStruct((N_NODES, F), jnp.float32),
    )(q, hws, dinv, b2)


def kernel(x, edge_index, W1, b1, W2, b2):
    src = edge_index[0].astype(jnp.int32)
    dst = edge_index[1].astype(jnp.int32)
    x_pad = jnp.pad(x, ((0, NPAD - N_NODES), (0, 0)))
    src_sc = src.reshape(NW, KSEG, CCH)
    dst_sc = dst.reshape(NW, KSEG, CCH)

    degp = _sc_deg(dst_sc)
    xs, dinv = _tc_prescale(degp, x_pad)
    p = _sc_segsum(src_sc, dst_sc, xs)
    h, hws = _tc_matmul(p, xs, dinv, W1, b1.reshape(1, HID), W2)
    q = _sc_segsum(src_sc, dst_sc, hws)
    logits = _tc_final(q, hws, dinv, b2.reshape(1, F))
    return h, logits
